# baseline (speedup 1.0000x reference)
import jax
import jax.numpy as jnp
from jax.experimental import pallas as pl
from jax.experimental.pallas import tpu as pltpu

E = 16
K = 2
D = 1024
F = 512
T = 1024
B = 128
NJ = T // B


def _moe_body(x_ref, gate_ref, w13_ref, w2_ref, out_ref, rank_ref, comb_ref):
    e = pl.program_id(0)
    j = pl.program_id(1)

    @pl.when((e == 0) & (j == 0))
    def _routing():
        x = x_ref[...]
        logits = jax.lax.dot_general(
            gate_ref[...], x, (((1,), (1,)), ((), ())),
            preferred_element_type=jnp.float32)
        ii = jax.lax.broadcasted_iota(jnp.int32, (E, T), 0)
        m1 = jnp.max(logits, axis=0, keepdims=True)
        i1 = jnp.min(jnp.where(logits == m1, ii, E), axis=0, keepdims=True)
        masked = jnp.where(ii == i1, -jnp.inf, logits)
        m2 = jnp.max(masked, axis=0, keepdims=True)
        i2 = jnp.min(jnp.where(masked == m2, ii, E), axis=0, keepdims=True)
        dd = jnp.exp(m2 - m1)
        w1 = 1.0 / (1.0 + dd)
        w2 = dd / (1.0 + dd)
        sel1 = ii == i1
        sel2 = ii == i2
        comb_ref[...] = jnp.where(sel1, w1, 0.0) + jnp.where(sel2, w2, 0.0)
        mask = (sel1 | sel2).astype(jnp.float32)
        ta = jax.lax.broadcasted_iota(jnp.int32, (T, T), 0)
        tb = jax.lax.broadcasted_iota(jnp.int32, (T, T), 1)
        lt = (ta < tb).astype(jnp.float32)
        rank = jax.lax.dot_general(mask, lt, (((1,), (0,)), ((), ())),
                                   preferred_element_type=jnp.float32)
        rank_ref[...] = jnp.where(mask > 0.0, rank, 2.0 * T)
        out_ref[...] = jnp.zeros_like(out_ref)

    rank_row = rank_ref[pl.ds(e, 1), :]
    count = jnp.sum(jnp.where(rank_row < 2.0 * T, 1.0, 0.0))

    @pl.when(j * B < count)
    def _block():
        x = x_ref[...]
        svec = (j * B + jax.lax.broadcasted_iota(jnp.int32, (B, 1), 0)
                ).astype(jnp.float32)
        G = (rank_row == svec).astype(jnp.float32)
        rows = jax.lax.dot_general(G, x, (((1,), (0,)), ((), ())),
                                   preferred_element_type=jnp.float32)
        w13 = w13_ref[0]
        g = jax.lax.dot_general(rows, w13[:F, :], (((1,), (1,)), ((), ())),
                                preferred_element_type=jnp.float32)
        u = jax.lax.dot_general(rows, w13[F:, :], (((1,), (1,)), ((), ())),
                                preferred_element_type=jnp.float32)
        act = g / (1.0 + jnp.exp(-g)) * u
        o = jax.lax.dot_general(act, w2_ref[0], (((1,), (1,)), ((), ())),
                                preferred_element_type=jnp.float32)
        GW = G * comb_ref[pl.ds(e, 1), :]
        out_ref[...] += jax.lax.dot_general(
            GW, o, (((0,), (0,)), ((), ())),
            preferred_element_type=jnp.float32)


@jax.jit
def kernel(hidden_states, gate_weight, w13_weight, w2_weight):
    return pl.pallas_call(
        _moe_body,
        grid=(E, NJ),
        in_specs=[
            pl.BlockSpec((T, D), lambda e, j: (0, 0)),
            pl.BlockSpec((E, D), lambda e, j: (0, 0)),
            pl.BlockSpec((1, 2 * F, D), lambda e, j: (e, 0, 0)),
            pl.BlockSpec((1, D, F), lambda e, j: (e, 0, 0)),
        ],
        out_specs=pl.BlockSpec((T, D), lambda e, j: (0, 0)),
        out_shape=jax.ShapeDtypeStruct((T, D), jnp.float32),
        scratch_shapes=[
            pltpu.VMEM((E, T), jnp.float32),
            pltpu.VMEM((E, T), jnp.float32),
        ],
        compiler_params=pltpu.CompilerParams(
            dimension_semantics=("arbitrary", "arbitrary"),
        ),
    )(hidden_states, gate_weight, w13_weight, w2_weight)


# speedup vs baseline: 1.6613x; 1.6613x over previous
import jax
import jax.numpy as jnp
from jax.experimental import pallas as pl
from jax.experimental.pallas import tpu as pltpu

E = 16
K = 2
D = 1024
F = 512
T = 1024
B = 128
NJ = T // B


def _moe_body(x_ref, gate_ref, w13_ref, w2_ref, out_ref, rank_ref, comb_ref):
    e = pl.program_id(0)

    @pl.when(e == 0)
    def _routing():
        x = x_ref[...]
        logits = jax.lax.dot_general(
            gate_ref[...], x, (((1,), (1,)), ((), ())),
            preferred_element_type=jnp.float32)
        ii = jax.lax.broadcasted_iota(jnp.int32, (E, T), 0)
        m1 = jnp.max(logits, axis=0, keepdims=True)
        i1 = jnp.min(jnp.where(logits == m1, ii, E), axis=0, keepdims=True)
        masked = jnp.where(ii == i1, -jnp.inf, logits)
        m2 = jnp.max(masked, axis=0, keepdims=True)
        i2 = jnp.min(jnp.where(masked == m2, ii, E), axis=0, keepdims=True)
        dd = jnp.exp(m2 - m1)
        w1 = 1.0 / (1.0 + dd)
        w2 = dd / (1.0 + dd)
        sel1 = ii == i1
        sel2 = ii == i2
        comb_ref[...] = jnp.where(sel1, w1, 0.0) + jnp.where(sel2, w2, 0.0)
        mask = (sel1 | sel2).astype(jnp.float32)
        ta = jax.lax.broadcasted_iota(jnp.int32, (T, T), 0)
        tb = jax.lax.broadcasted_iota(jnp.int32, (T, T), 1)
        lt = (ta < tb).astype(jnp.float32)
        rank = jax.lax.dot_general(mask, lt, (((1,), (0,)), ((), ())),
                                   preferred_element_type=jnp.float32)
        rank_ref[...] = jnp.where(mask > 0.0, rank, 2.0 * T)
        out_ref[...] = jnp.zeros_like(out_ref)

    rank_row = rank_ref[pl.ds(e, 1), :]
    count = jnp.sum(jnp.where(rank_row < 2.0 * T, 1.0, 0.0)).astype(jnp.int32)
    nb = (count + (B - 1)) // B
    comb_row = comb_ref[pl.ds(e, 1), :]
    x = x_ref[...]
    w13 = w13_ref[0]
    w2w = w2_ref[0]

    def _block(j, _):
        svec = (j * B + jax.lax.broadcasted_iota(jnp.int32, (B, 1), 0)
                ).astype(jnp.float32)
        G = (rank_row == svec).astype(jnp.float32)
        rows = jax.lax.dot_general(G, x, (((1,), (0,)), ((), ())),
                                   preferred_element_type=jnp.float32)
        g = jax.lax.dot_general(rows, w13[:F, :], (((1,), (1,)), ((), ())),
                                preferred_element_type=jnp.float32)
        u = jax.lax.dot_general(rows, w13[F:, :], (((1,), (1,)), ((), ())),
                                preferred_element_type=jnp.float32)
        act = g / (1.0 + jnp.exp(-g)) * u
        o = jax.lax.dot_general(act, w2w, (((1,), (1,)), ((), ())),
                                preferred_element_type=jnp.float32)
        GW = G * comb_row
        out_ref[...] += jax.lax.dot_general(
            GW, o, (((0,), (0,)), ((), ())),
            preferred_element_type=jnp.float32)
        return _

    jax.lax.fori_loop(0, nb, _block, None)


@jax.jit
def kernel(hidden_states, gate_weight, w13_weight, w2_weight):
    return pl.pallas_call(
        _moe_body,
        grid=(E,),
        in_specs=[
            pl.BlockSpec((T, D), lambda e: (0, 0)),
            pl.BlockSpec((E, D), lambda e: (0, 0)),
            pl.BlockSpec((1, 2 * F, D), lambda e: (e, 0, 0)),
            pl.BlockSpec((1, D, F), lambda e: (e, 0, 0)),
        ],
        out_specs=pl.BlockSpec((T, D), lambda e: (0, 0)),
        out_shape=jax.ShapeDtypeStruct((T, D), jnp.float32),
        scratch_shapes=[
            pltpu.VMEM((E, T), jnp.float32),
            pltpu.VMEM((E, T), jnp.float32),
        ],
        compiler_params=pltpu.CompilerParams(
            dimension_semantics=("arbitrary",),
        ),
    )(hidden_states, gate_weight, w13_weight, w2_weight)


# speedup vs baseline: 1.8423x; 1.1089x over previous
import jax
import jax.numpy as jnp
from jax.experimental import pallas as pl
from jax.experimental.pallas import tpu as pltpu

E = 16
K = 2
D = 1024
F = 512
T = 1024
B = 128
NJ = T // B


def _moe_body(x_ref, gate_ref, w13_ref, w2_ref, out_ref, rank_ref, comb_ref):
    e = pl.program_id(0)

    @pl.when(e == 0)
    def _routing():
        x = x_ref[...]
        logits = jax.lax.dot_general(
            gate_ref[...], x, (((1,), (1,)), ((), ())),
            preferred_element_type=jnp.float32)
        ii = jax.lax.broadcasted_iota(jnp.int32, (E, T), 0)
        m1 = jnp.max(logits, axis=0, keepdims=True)
        i1 = jnp.min(jnp.where(logits == m1, ii, E), axis=0, keepdims=True)
        masked = jnp.where(ii == i1, -jnp.inf, logits)
        m2 = jnp.max(masked, axis=0, keepdims=True)
        i2 = jnp.min(jnp.where(masked == m2, ii, E), axis=0, keepdims=True)
        dd = jnp.exp(m2 - m1)
        w1 = 1.0 / (1.0 + dd)
        w2 = dd / (1.0 + dd)
        sel1 = ii == i1
        sel2 = ii == i2
        comb_ref[...] = jnp.where(sel1, w1, 0.0) + jnp.where(sel2, w2, 0.0)
        mask = (sel1 | sel2).astype(jnp.float32)
        ta = jax.lax.broadcasted_iota(jnp.int32, (T, T), 0)
        tb = jax.lax.broadcasted_iota(jnp.int32, (T, T), 1)
        lt = (ta < tb).astype(jnp.float32)
        rank = jax.lax.dot_general(mask, lt, (((1,), (0,)), ((), ())),
                                   preferred_element_type=jnp.float32)
        rank_ref[...] = jnp.where(mask > 0.0, rank, 2.0 * T)
        out_ref[...] = jnp.zeros_like(out_ref)

    rank_row = rank_ref[pl.ds(e, 1), :]
    count = jnp.sum(jnp.where(rank_row < 2.0 * T, 1.0, 0.0)).astype(jnp.int32)
    nb = (count + (B - 1)) // B

    def _block(j, _):
        svec = (j * B + jax.lax.broadcasted_iota(jnp.int32, (B, 1), 0)
                ).astype(jnp.float32)
        G = (rank_ref[pl.ds(e, 1), :] == svec).astype(jnp.float32)
        rows = jax.lax.dot_general(G, x_ref[...], (((1,), (0,)), ((), ())),
                                   preferred_element_type=jnp.float32)
        g = jax.lax.dot_general(rows, w13_ref[0, :F, :], (((1,), (1,)), ((), ())),
                                preferred_element_type=jnp.float32)
        u = jax.lax.dot_general(rows, w13_ref[0, F:, :], (((1,), (1,)), ((), ())),
                                preferred_element_type=jnp.float32)
        act = g / (1.0 + jnp.exp(-g)) * u
        o = jax.lax.dot_general(act, w2_ref[0], (((1,), (1,)), ((), ())),
                                preferred_element_type=jnp.float32)
        GW = G * comb_ref[pl.ds(e, 1), :]
        out_ref[...] += jax.lax.dot_general(
            GW, o, (((0,), (0,)), ((), ())),
            preferred_element_type=jnp.float32)
        return _

    jax.lax.fori_loop(0, nb, _block, None)


@jax.jit
def kernel(hidden_states, gate_weight, w13_weight, w2_weight):
    return pl.pallas_call(
        _moe_body,
        grid=(E,),
        in_specs=[
            pl.BlockSpec((T, D), lambda e: (0, 0)),
            pl.BlockSpec((E, D), lambda e: (0, 0)),
            pl.BlockSpec((1, 2 * F, D), lambda e: (e, 0, 0)),
            pl.BlockSpec((1, D, F), lambda e: (e, 0, 0)),
        ],
        out_specs=pl.BlockSpec((T, D), lambda e: (0, 0)),
        out_shape=jax.ShapeDtypeStruct((T, D), jnp.float32),
        scratch_shapes=[
            pltpu.VMEM((E, T), jnp.float32),
            pltpu.VMEM((E, T), jnp.float32),
        ],
        compiler_params=pltpu.CompilerParams(
            dimension_semantics=("arbitrary",),
        ),
    )(hidden_states, gate_weight, w13_weight, w2_weight)
